# baseline (speedup 1.0000x reference)
import functools

import jax
import jax.numpy as jnp
from jax import lax
from jax.experimental import pallas as pl
from jax.experimental.pallas import tpu as pltpu
from jax.experimental.pallas import tpu_sc as plsc

_L = 16


def _make_dot(B, V, D, nc, ns):
    nw = nc * ns
    bpw = B // nw
    nd = D // _L
    ng = bpw // _L

    mesh = plsc.VectorSubcoreMesh(core_axis_name="c", subcore_axis_name="s")

    @functools.partial(
        pl.kernel,
        out_type=jax.ShapeDtypeStruct((B,), jnp.float32),
        mesh=mesh,
        compiler_params=pltpu.CompilerParams(
            needs_layout_passes=False, use_tc_tiling_on_sc=False),
        scratch_types=[
            pltpu.VMEM((bpw,), jnp.int32),
            pltpu.VMEM((bpw,), jnp.int32),
            pltpu.VMEM((bpw, D), jnp.float32),
            pltpu.VMEM((bpw, D), jnp.float32),
            pltpu.VMEM((_L, _L), jnp.float32),
            pltpu.VMEM((bpw,), jnp.float32),
            pltpu.SemaphoreType.DMA,
        ],
    )
    def dot(left_h, right_h, lemb_h, remb_h, out_h,
            idx_l, idx_r, l_rows, r_rows, acc_s, out_v, sem):
        wid = lax.axis_index("s") * nc + lax.axis_index("c")
        base = wid * bpw

        pltpu.sync_copy(left_h.at[pl.ds(base, bpw)], idx_l)
        pltpu.sync_copy(right_h.at[pl.ds(base, bpw)], idx_r)

        cp1 = pltpu.async_copy(lemb_h.at[idx_l], l_rows, sem)
        cp2 = pltpu.async_copy(remb_h.at[idx_r], r_rows, sem)
        cp1.wait()
        cp2.wait()

        lane = lax.iota(jnp.int32, _L)

        def group(g, carry):
            p0 = g * _L
            for j in range(_L):
                p = p0 + j
                acc = l_rows[p, pl.ds(0, _L)] * r_rows[p, pl.ds(0, _L)]
                for c in range(1, nd):
                    acc = acc + (l_rows[p, pl.ds(c * _L, _L)]
                                 * r_rows[p, pl.ds(c * _L, _L)])
                acc_s[j, pl.ds(0, _L)] = acc
            tot = plsc.load_gather(
                acc_s, [lane, jnp.zeros((_L,), jnp.int32)])
            for d in range(1, _L):
                tot = tot + plsc.load_gather(
                    acc_s, [lane, jnp.full((_L,), d, jnp.int32)])
            out_v[pl.ds(p0, _L)] = tot
            return carry

        lax.fori_loop(0, ng, group, 0)

        pltpu.sync_copy(out_v, out_h.at[pl.ds(base, bpw)])

    return dot


def _make_bias(B, V, nc, ns):
    nw = nc * ns
    bpw = B // nw
    ng = bpw // _L

    mesh = plsc.VectorSubcoreMesh(core_axis_name="c", subcore_axis_name="s")

    @functools.partial(
        pl.kernel,
        out_type=jax.ShapeDtypeStruct((nw, 8, 128), jnp.float32),
        mesh=mesh,
        compiler_params=pltpu.CompilerParams(
            needs_layout_passes=False, use_tc_tiling_on_sc=True),
        scratch_types=[
            pltpu.VMEM((1024,), jnp.int32),
            pltpu.VMEM((1024,), jnp.int32),
            pltpu.VMEM((bpw,), jnp.float32),
            pltpu.VMEM((bpw,), jnp.float32),
            pltpu.VMEM((8, 128), jnp.float32),
            pltpu.SemaphoreType.DMA,
        ],
    )
    def bias(left_h, right_h, lb_h, rb_h, out_h,
             lidx, ridx, bias_l, bias_r, out_v, sem):
        wid = lax.axis_index("s") * nc + lax.axis_index("c")
        win = (wid // 2) * 1024
        sub = (wid % 2) * bpw

        pltpu.sync_copy(left_h.at[pl.ds(win, 1024)], lidx)
        pltpu.sync_copy(right_h.at[pl.ds(win, 1024)], ridx)

        cp1 = pltpu.async_copy(lb_h.at[lidx.at[pl.ds(sub, bpw)]], bias_l, sem)
        cp2 = pltpu.async_copy(rb_h.at[ridx.at[pl.ds(sub, bpw)]], bias_r, sem)
        cp1.wait()
        cp2.wait()

        def group(g, carry):
            p0 = g * _L
            tot = bias_l[pl.ds(p0, _L)] + bias_r[pl.ds(p0, _L)]
            out_v[p0 // 128, pl.ds(p0 % 128, _L)] = tot
            return carry

        lax.fori_loop(0, ng, group, 0)

        pltpu.sync_copy(out_v, out_h.at[wid])

    return bias


def kernel(left, right, l_emb, l_bias, r_emb, r_bias):
    B = left.shape[0]
    V, D = l_emb.shape
    info = plsc.get_sparse_core_info()
    nw = info.num_cores * info.num_subcores
    bpw = B // nw
    li = left.astype(jnp.int32)
    ri = right.astype(jnp.int32)
    dots = _make_dot(B, V, D, info.num_cores, info.num_subcores)(
        li, ri, l_emb, r_emb)
    bias3 = _make_bias(B, V, info.num_cores, info.num_subcores)(
        li, ri, jnp.squeeze(l_bias, 1), jnp.squeeze(r_bias, 1))
    biases = bias3.reshape(nw, 1024)[:, :bpw].reshape(B)
    return dots + biases


# speedup vs baseline: 1.0017x; 1.0017x over previous
import functools

import jax
import jax.numpy as jnp
from jax import lax
from jax.experimental import pallas as pl
from jax.experimental.pallas import tpu as pltpu
from jax.experimental.pallas import tpu_sc as plsc

_L = 16


def _make_glove(B, V, D, nc, ns):
    nw = nc * ns
    assert B % nw == 0
    bpw = B // nw
    assert D % _L == 0
    nd = D // _L
    ng = bpw // _L

    mesh = plsc.VectorSubcoreMesh(core_axis_name="c", subcore_axis_name="s")

    @functools.partial(
        pl.kernel,
        out_type=jax.ShapeDtypeStruct((B,), jnp.float32),
        mesh=mesh,
        compiler_params=pltpu.CompilerParams(
            needs_layout_passes=False, use_tc_tiling_on_sc=False),
        scratch_types=[
            pltpu.VMEM((bpw,), jnp.int32),
            pltpu.VMEM((bpw,), jnp.int32),
            pltpu.VMEM((bpw, D), jnp.float32),
            pltpu.VMEM((bpw, D), jnp.float32),
            pltpu.VMEM((bpw,), jnp.float32),
            pltpu.VMEM((bpw,), jnp.float32),
            pltpu.VMEM((_L, _L), jnp.float32),
            pltpu.VMEM((bpw,), jnp.float32),
            pltpu.SemaphoreType.DMA,
        ],
    )
    def glove(left_h, right_h, lemb_h, lbias_h, remb_h, rbias_h, out_h,
              idx_l, idx_r, l_rows, r_rows, bias_l, bias_r, acc_s, out_v, sem):
        wid = lax.axis_index("s") * nc + lax.axis_index("c")
        base = wid * bpw

        pltpu.sync_copy(left_h.at[pl.ds(base, bpw)], idx_l)
        pltpu.sync_copy(right_h.at[pl.ds(base, bpw)], idx_r)

        cps = [
            pltpu.async_copy(lemb_h.at[idx_l], l_rows, sem),
            pltpu.async_copy(remb_h.at[idx_r], r_rows, sem),
            pltpu.async_copy(lbias_h.at[idx_l], bias_l, sem),
            pltpu.async_copy(rbias_h.at[idx_r], bias_r, sem),
        ]
        for cp in cps:
            cp.wait()

        lane = lax.iota(jnp.int32, _L)

        def group(g, carry):
            p0 = g * _L
            for j in range(_L):
                p = p0 + j
                acc = l_rows[p, pl.ds(0, _L)] * r_rows[p, pl.ds(0, _L)]
                for c in range(1, nd):
                    acc = acc + (l_rows[p, pl.ds(c * _L, _L)]
                                 * r_rows[p, pl.ds(c * _L, _L)])
                acc_s[j, pl.ds(0, _L)] = acc
            tot = bias_l[pl.ds(p0, _L)] + bias_r[pl.ds(p0, _L)]
            for d in range(_L):
                tot = tot + plsc.load_gather(
                    acc_s, [lane, jnp.full((_L,), d, jnp.int32)])
            out_v[pl.ds(p0, _L)] = tot
            return carry

        lax.fori_loop(0, ng, group, 0)

        pltpu.sync_copy(out_v, out_h.at[pl.ds(base, bpw)])

    return glove


def kernel(left, right, l_emb, l_bias, r_emb, r_bias):
    B = left.shape[0]
    V, D = l_emb.shape
    info = plsc.get_sparse_core_info()
    fn = _make_glove(B, V, D, info.num_cores, info.num_subcores)
    return fn(
        left.astype(jnp.int32),
        right.astype(jnp.int32),
        l_emb,
        jnp.squeeze(l_bias, 1),
        r_emb,
        jnp.squeeze(r_bias, 1),
    )
